# Initial kernel scaffold; baseline (speedup 1.0000x reference)
import jax
import jax.numpy as jnp
from jax.experimental import pallas as pl


def kernel(h_dag, obs_ptr, W, b):
    raise NotImplementedError("write your pallas kernel here")


# speedup vs baseline: 12.4379x; 12.4379x over previous
import jax
import jax.numpy as jnp
from jax.experimental import pallas as pl


def _body(lo_ref, hi_ref, x_ref, wt_ref, b_ref, o_ref):
    i = pl.program_id(0)
    r = x_ref.shape[0]
    nseg = o_ref.shape[0]
    h = jnp.dot(x_ref[...], wt_ref[...], preferred_element_type=jnp.float32)
    h = h + b_ref[...]
    h = jnp.where(h > 0, h, 0.2 * h)
    pos = i * r + jax.lax.broadcasted_iota(jnp.int32, (nseg, r), 1)
    sel = ((pos >= lo_ref[...]) & (pos < hi_ref[...])).astype(jnp.float32)
    contrib = jnp.dot(sel, h, preferred_element_type=jnp.float32)

    @pl.when(i == 0)
    def _init():
        o_ref[...] = contrib

    @pl.when(i != 0)
    def _acc():
        o_ref[...] += contrib


def kernel(h_dag, obs_ptr, W, b):
    n, d = h_dag.shape
    nseg = obs_ptr.shape[0] - 1
    block_r = 2048
    lo = obs_ptr[:-1].reshape(nseg, 1)
    hi = obs_ptr[1:].reshape(nseg, 1)
    return pl.pallas_call(
        _body,
        grid=(n // block_r,),
        in_specs=[
            pl.BlockSpec((nseg, 1), lambda i: (0, 0)),
            pl.BlockSpec((nseg, 1), lambda i: (0, 0)),
            pl.BlockSpec((block_r, d), lambda i: (i, 0)),
            pl.BlockSpec((d, d), lambda i: (0, 0)),
            pl.BlockSpec((1, d), lambda i: (0, 0)),
        ],
        out_specs=pl.BlockSpec((nseg, d), lambda i: (0, 0)),
        out_shape=jax.ShapeDtypeStruct((nseg, d), jnp.float32),
    )(lo, hi, h_dag, W.T, b.reshape(1, d))
